# baseline (speedup 1.0000x reference)
import functools

import jax
import jax.numpy as jnp
from jax import lax
from jax.experimental import pallas as pl
from jax.experimental.pallas import tpu as pltpu


def _pick_h_blk(hidden):
    for blk in (8192, 4096, 2048, 1024, 512, 256, 128):
        if hidden % blk == 0:
            return blk
    return None


def _kstream_kernel(x_ref, w_ref, o_ref):
    k = pl.program_id(0)
    part = lax.dot_general(
        x_ref[...],
        w_ref[...],
        dimension_numbers=(((2,), (1,)), ((), ())),
        preferred_element_type=jnp.float32,
    )

    @pl.when(k == 0)
    def _():
        o_ref[...] = part

    @pl.when(k != 0)
    def _():
        o_ref[...] += part


def _single_shot(x_ref, w_ref, o_ref):
    o_ref[...] = lax.dot_general(
        x_ref[...],
        w_ref[...],
        dimension_numbers=(((2,), (1,)), ((), ())),
        preferred_element_type=jnp.float32,
    )


def _linear_impl(x, weight):
    batch, seq, hidden = x.shape
    out_features, hidden_w = weight.shape
    assert hidden == hidden_w
    m = batch * seq

    xt = jnp.transpose(x, (1, 0, 2))

    cost = pl.CostEstimate(
        flops=2 * m * out_features * hidden,
        transcendentals=0,
        bytes_accessed=(m * hidden + out_features * hidden) * 4
        + m * out_features * 4,
    )

    h_blk = _pick_h_blk(hidden)
    if h_blk is None or hidden // h_blk < 2:
        yt = pl.pallas_call(
            _single_shot,
            out_shape=jax.ShapeDtypeStruct((seq, batch, out_features), jnp.float32),
            in_specs=[
                pl.BlockSpec(memory_space=pltpu.MemorySpace.VMEM),
                pl.BlockSpec(memory_space=pltpu.MemorySpace.VMEM),
            ],
            out_specs=pl.BlockSpec(memory_space=pltpu.MemorySpace.VMEM),
            cost_estimate=cost,
        )(xt, weight)
        return jnp.transpose(yt, (1, 0, 2)).astype(x.dtype)

    vmem_limit = 60000 * 1024

    seq_blk = 4
    ni = seq // seq_blk
    yt = pl.pallas_call(
        _single_shot,
        out_shape=jax.ShapeDtypeStruct((seq, batch, out_features), jnp.float32),
        grid=(ni,),
        in_specs=[
            pl.BlockSpec((seq_blk, batch, hidden), lambda i: (i, 0, 0)),
            pl.BlockSpec((out_features, hidden), lambda i: (0, 0)),
        ],
        out_specs=pl.BlockSpec(
            (seq_blk, batch, out_features), lambda i: (i, 0, 0)
        ),
        compiler_params=pltpu.CompilerParams(
            dimension_semantics=("arbitrary",),
            vmem_limit_bytes=vmem_limit,
        ),
        cost_estimate=cost,
    )(xt, weight)
    return jnp.transpose(yt, (1, 0, 2)).astype(x.dtype)


kernel = jax.jit(_linear_impl)


# speedup vs baseline: 1.4342x; 1.4342x over previous
import functools

import jax
import jax.numpy as jnp
from jax import lax
from jax.experimental import pallas as pl
from jax.experimental.pallas import tpu as pltpu


def _pick_h_blk(hidden):
    for blk in (8192, 4096, 2048, 1024, 512, 256, 128):
        if hidden % blk == 0:
            return blk
    return None


def _kstream_kernel(x_ref, w_ref, o_ref):
    k = pl.program_id(0)
    part = lax.dot_general(
        x_ref[...],
        w_ref[...],
        dimension_numbers=(((2,), (1,)), ((), ())),
        preferred_element_type=jnp.float32,
    )

    @pl.when(k == 0)
    def _():
        o_ref[...] = part

    @pl.when(k != 0)
    def _():
        o_ref[...] += part


def _kstream_kernel2(x_ref, w0_ref, w1_ref, o_ref):
    k = pl.program_id(0)
    x = x_ref[...]
    n_half = w0_ref.shape[0]
    p0 = lax.dot_general(
        x, w0_ref[...],
        dimension_numbers=(((2,), (1,)), ((), ())),
        preferred_element_type=jnp.float32,
    )
    p1 = lax.dot_general(
        x, w1_ref[...],
        dimension_numbers=(((2,), (1,)), ((), ())),
        preferred_element_type=jnp.float32,
    )

    @pl.when(k == 0)
    def _():
        o_ref[:, :, :n_half] = p0
        o_ref[:, :, n_half:] = p1

    @pl.when(k != 0)
    def _():
        o_ref[:, :, :n_half] += p0
        o_ref[:, :, n_half:] += p1


def _single_shot(x_ref, w_ref, o_ref):
    o_ref[...] = lax.dot_general(
        x_ref[...],
        w_ref[...],
        dimension_numbers=(((2,), (1,)), ((), ())),
        preferred_element_type=jnp.float32,
    )


def _linear_impl(x, weight):
    batch, seq, hidden = x.shape
    out_features, hidden_w = weight.shape
    assert hidden == hidden_w
    m = batch * seq

    xt = jnp.transpose(x, (1, 0, 2))

    cost = pl.CostEstimate(
        flops=2 * m * out_features * hidden,
        transcendentals=0,
        bytes_accessed=(m * hidden + out_features * hidden) * 4
        + m * out_features * 4,
    )

    h_blk = _pick_h_blk(hidden)
    if h_blk is None or hidden // h_blk < 2:
        yt = pl.pallas_call(
            _single_shot,
            out_shape=jax.ShapeDtypeStruct((seq, batch, out_features), jnp.float32),
            in_specs=[
                pl.BlockSpec(memory_space=pltpu.MemorySpace.VMEM),
                pl.BlockSpec(memory_space=pltpu.MemorySpace.VMEM),
            ],
            out_specs=pl.BlockSpec(memory_space=pltpu.MemorySpace.VMEM),
            cost_estimate=cost,
        )(xt, weight)
        return jnp.transpose(yt, (1, 0, 2)).astype(x.dtype)

    vmem_limit = 60000 * 1024

    nk = hidden // h_blk
    if out_features % 2 == 0 and out_features // 2 >= 8:
        n_half = out_features // 2
        yt = pl.pallas_call(
            _kstream_kernel2,
            out_shape=jax.ShapeDtypeStruct(
                (seq, batch, out_features), jnp.float32
            ),
            grid=(nk,),
            in_specs=[
                pl.BlockSpec((seq, batch, h_blk), lambda k: (0, 0, k)),
                pl.BlockSpec((n_half, h_blk), lambda k: (0, k)),
                pl.BlockSpec((n_half, h_blk), lambda k: (1, k)),
            ],
            out_specs=pl.BlockSpec(
                (seq, batch, out_features), lambda k: (0, 0, 0)
            ),
            compiler_params=pltpu.CompilerParams(
                dimension_semantics=("arbitrary",),
                vmem_limit_bytes=vmem_limit,
            ),
            cost_estimate=cost,
        )(xt, weight, weight)
        return jnp.transpose(yt, (1, 0, 2)).astype(x.dtype)

    yt = pl.pallas_call(
        _kstream_kernel,
        out_shape=jax.ShapeDtypeStruct((seq, batch, out_features), jnp.float32),
        grid=(nk,),
        in_specs=[
            pl.BlockSpec((seq, batch, h_blk), lambda k: (0, 0, k)),
            pl.BlockSpec((out_features, h_blk), lambda k: (0, k)),
        ],
        out_specs=pl.BlockSpec((seq, batch, out_features), lambda k: (0, 0, 0)),
        compiler_params=pltpu.CompilerParams(
            dimension_semantics=("arbitrary",),
            vmem_limit_bytes=vmem_limit,
        ),
        cost_estimate=cost,
    )(xt, weight)
    return jnp.transpose(yt, (1, 0, 2)).astype(x.dtype)


kernel = jax.jit(_linear_impl)


# speedup vs baseline: 1.4422x; 1.0056x over previous
import functools

import jax
import jax.numpy as jnp
from jax import lax
from jax.experimental import pallas as pl
from jax.experimental.pallas import tpu as pltpu


def _pick_h_blk(hidden):
    for blk in (8192, 4096, 2048, 1024, 512, 256, 128):
        if hidden % blk == 0:
            return blk
    return None


def _kstream_kernel(x_ref, w_ref, o_ref):
    k = pl.program_id(0)
    seq, batch, h_blk = x_ref.shape
    x2 = x_ref[...].reshape(seq * batch, h_blk)
    part = lax.dot_general(
        x2,
        w_ref[...],
        dimension_numbers=(((1,), (1,)), ((), ())),
        preferred_element_type=jnp.float32,
    ).reshape(seq, batch, -1)

    @pl.when(k == 0)
    def _():
        o_ref[...] = part

    @pl.when(k != 0)
    def _():
        o_ref[...] += part


def _single_shot(x_ref, w_ref, o_ref):
    o_ref[...] = lax.dot_general(
        x_ref[...],
        w_ref[...],
        dimension_numbers=(((2,), (1,)), ((), ())),
        preferred_element_type=jnp.float32,
    )


def _linear_impl(x, weight):
    batch, seq, hidden = x.shape
    out_features, hidden_w = weight.shape
    assert hidden == hidden_w
    m = batch * seq

    xt = jnp.transpose(x, (1, 0, 2))

    cost = pl.CostEstimate(
        flops=2 * m * out_features * hidden,
        transcendentals=0,
        bytes_accessed=(m * hidden + out_features * hidden) * 4
        + m * out_features * 4,
    )

    h_blk = _pick_h_blk(hidden)
    if h_blk is None or hidden // h_blk < 2:
        yt = pl.pallas_call(
            _single_shot,
            out_shape=jax.ShapeDtypeStruct((seq, batch, out_features), jnp.float32),
            in_specs=[
                pl.BlockSpec(memory_space=pltpu.MemorySpace.VMEM),
                pl.BlockSpec(memory_space=pltpu.MemorySpace.VMEM),
            ],
            out_specs=pl.BlockSpec(memory_space=pltpu.MemorySpace.VMEM),
            cost_estimate=cost,
        )(xt, weight)
        return jnp.transpose(yt, (1, 0, 2)).astype(x.dtype)

    vmem_limit = 60000 * 1024

    nk = hidden // h_blk
    yt = pl.pallas_call(
        _kstream_kernel,
        out_shape=jax.ShapeDtypeStruct((seq, batch, out_features), jnp.float32),
        grid=(nk,),
        in_specs=[
            pl.BlockSpec((seq, batch, h_blk), lambda k: (0, 0, k)),
            pl.BlockSpec((out_features, h_blk), lambda k: (0, k)),
        ],
        out_specs=pl.BlockSpec((seq, batch, out_features), lambda k: (0, 0, 0)),
        compiler_params=pltpu.CompilerParams(
            dimension_semantics=("arbitrary",),
            vmem_limit_bytes=vmem_limit,
        ),
        cost_estimate=cost,
    )(xt, weight)
    return jnp.transpose(yt, (1, 0, 2)).astype(x.dtype)


kernel = jax.jit(_linear_impl)


# speedup vs baseline: 1.4688x; 1.0185x over previous
import jax
import jax.numpy as jnp
from jax import lax
from jax.experimental import pallas as pl
from jax.experimental.pallas import tpu as pltpu


def _pick_h_blk(hidden):
    for blk in (8192, 4096, 2048, 1024, 512, 256, 128):
        if hidden % blk == 0:
            return blk
    return None


def _kstream_kernel(x_ref, w_ref, o_ref):
    k = pl.program_id(0)
    part = lax.dot_general(
        x_ref[...],
        w_ref[...],
        dimension_numbers=(((2,), (1,)), ((), ())),
        preferred_element_type=jnp.float32,
    )

    @pl.when(k == 0)
    def _():
        o_ref[...] = part

    @pl.when(k != 0)
    def _():
        o_ref[...] += part


def _single_shot(x_ref, w_ref, o_ref):
    o_ref[...] = lax.dot_general(
        x_ref[...],
        w_ref[...],
        dimension_numbers=(((2,), (1,)), ((), ())),
        preferred_element_type=jnp.float32,
    )


def _linear_impl(x, weight):
    batch, seq, hidden = x.shape
    out_features, hidden_w = weight.shape
    assert hidden == hidden_w
    m = batch * seq

    xt = jnp.transpose(x, (1, 0, 2))

    cost = pl.CostEstimate(
        flops=2 * m * out_features * hidden,
        transcendentals=0,
        bytes_accessed=(m * hidden + out_features * hidden) * 4
        + m * out_features * 4,
    )

    h_blk = _pick_h_blk(hidden)
    if h_blk is None or hidden // h_blk < 2:
        yt = pl.pallas_call(
            _single_shot,
            out_shape=jax.ShapeDtypeStruct((seq, batch, out_features), jnp.float32),
            in_specs=[
                pl.BlockSpec(memory_space=pltpu.MemorySpace.VMEM),
                pl.BlockSpec(memory_space=pltpu.MemorySpace.VMEM),
            ],
            out_specs=pl.BlockSpec(memory_space=pltpu.MemorySpace.VMEM),
            cost_estimate=cost,
        )(xt, weight)
        return jnp.transpose(yt, (1, 0, 2)).astype(x.dtype)

    vmem_limit = 60000 * 1024

    nk = hidden // h_blk
    yt = pl.pallas_call(
        _kstream_kernel,
        out_shape=jax.ShapeDtypeStruct((seq, batch, out_features), jnp.float32),
        grid=(nk,),
        in_specs=[
            pl.BlockSpec((seq, batch, h_blk), lambda k: (0, 0, k)),
            pl.BlockSpec((out_features, h_blk), lambda k: (0, k)),
        ],
        out_specs=pl.BlockSpec((seq, batch, out_features), lambda k: (0, 0, 0)),
        compiler_params=pltpu.CompilerParams(
            dimension_semantics=("arbitrary",),
            vmem_limit_bytes=vmem_limit,
        ),
        cost_estimate=cost,
    )(xt, weight)
    return jnp.transpose(yt, (1, 0, 2)).astype(x.dtype)


kernel = jax.jit(_linear_impl)
